# Initial kernel scaffold; baseline (speedup 1.0000x reference)
import jax
import jax.numpy as jnp
from jax.experimental import pallas as pl


def kernel(entity_embeddings, weight, edge_index, edge_type):
    raise NotImplementedError("write your pallas kernel here")


# speedup vs baseline: 21.4743x; 21.4743x over previous
import functools

import jax
import jax.numpy as jnp
from jax import lax
from jax.experimental import pallas as pl
from jax.experimental.pallas import tpu as pltpu
from jax.experimental.pallas import tpu_sc as plsc

N = 10000
EDG = 320000
F = 128
R = 8

NC, NS = 2, 16
H = N // 2
K = 128
CH = 158
EPT = K * CH
ETOT = NS * EPT
NACC = 5120
NDUMP = NACC - H
RPT = NACC // NS

BN = 1000
BF = 1000


def _mm_body(e_ref, w_ref, o_ref):
    o_ref[0] = jnp.dot(e_ref[...], w_ref[0], preferred_element_type=jnp.float32)


def _finish_body(p_ref, o_ref):
    o_ref[...] = jnp.maximum(p_ref[0], 0.0)


_mesh = plsc.VectorSubcoreMesh(
    core_axis_name="c", subcore_axis_name="s", num_cores=NC, num_subcores=NS
)


@functools.partial(
    pl.kernel,
    out_type=jax.ShapeDtypeStruct((NC, NACC, F), jnp.float32),
    mesh=_mesh,
    scratch_types=[
        pltpu.VMEM((CH, K), jnp.int32),
        pltpu.VMEM((CH, K), jnp.int32),
        pltpu.VMEM((K, F), jnp.float32),
        pltpu.VMEM((K, F), jnp.float32),
        pltpu.VMEM_SHARED((NACC, F), jnp.float32),
        pltpu.SemaphoreType.DMA,
        pltpu.SemaphoreType.DMA,
    ],
)
def _sc_scatter(t_hbm, fidx_hbm, tgt_hbm, zrows_hbm, out_hbm,
                fidx_v, tgt_v, rows_a, rows_b, accum, sem_a, sem_b):
    c = lax.axis_index("c")
    s = lax.axis_index("s")

    pltpu.sync_copy(zrows_hbm, accum.at[pl.ds(s * RPT, RPT)])
    plsc.subcore_barrier()

    pltpu.sync_copy(fidx_hbm.at[s], fidx_v)
    pltpu.sync_copy(tgt_hbm.at[c, s], tgt_v)

    pltpu.async_copy(t_hbm.at[fidx_v.at[0]], rows_a, sem_a)
    pltpu.async_copy(t_hbm.at[fidx_v.at[1]], rows_b, sem_b)

    def body(i, carry):
        ca = 2 * i
        cb = ca + 1
        pltpu.make_async_copy(t_hbm.at[fidx_v.at[ca]], rows_a, sem_a).wait()
        pltpu.sync_copy(rows_a, accum.at[tgt_v.at[ca]], add=True)
        pltpu.async_copy(t_hbm.at[fidx_v.at[ca + 2]], rows_a, sem_a)
        pltpu.make_async_copy(t_hbm.at[fidx_v.at[cb]], rows_b, sem_b).wait()
        pltpu.sync_copy(rows_b, accum.at[tgt_v.at[cb]], add=True)
        pltpu.async_copy(t_hbm.at[fidx_v.at[cb + 2]], rows_b, sem_b)
        return carry

    lax.fori_loop(0, CH // 2 - 1, body, 0)

    ca = CH - 2
    pltpu.make_async_copy(t_hbm.at[fidx_v.at[ca]], rows_a, sem_a).wait()
    pltpu.sync_copy(rows_a, accum.at[tgt_v.at[ca]], add=True)
    pltpu.make_async_copy(t_hbm.at[fidx_v.at[ca + 1]], rows_b, sem_b).wait()
    pltpu.sync_copy(rows_b, accum.at[tgt_v.at[ca + 1]], add=True)

    plsc.subcore_barrier()
    pltpu.sync_copy(accum.at[pl.ds(s * RPT, RPT)],
                    out_hbm.at[c, pl.ds(s * RPT, RPT)])


def kernel(entity_embeddings, weight, edge_index, edge_type):
    src = edge_index[0]
    tgt = edge_index[1]
    flat_idx = edge_type * N + src

    dump = H + (jnp.arange(EDG, dtype=jnp.int32) % NDUMP)
    tgt_lo = jnp.where(tgt < H, tgt, dump)
    tgt_hi = jnp.where(tgt >= H, tgt - H, dump)

    pad = ETOT - EDG
    fidx = jnp.concatenate(
        [flat_idx, jnp.zeros((pad,), jnp.int32)]).reshape(NS, CH, K)
    tgtp = jnp.concatenate([
        jnp.concatenate([tgt_lo, jnp.full((pad,), H, jnp.int32)]),
        jnp.concatenate([tgt_hi, jnp.full((pad,), H, jnp.int32)]),
    ]).reshape(NC, NS, CH, K)
    zrows = jnp.zeros((RPT, F), jnp.float32)

    t = pl.pallas_call(
        _mm_body,
        grid=(N // BN, R),
        in_specs=[
            pl.BlockSpec((BN, F), lambda i, r: (i, 0)),
            pl.BlockSpec((1, F, F), lambda i, r: (r, 0, 0)),
        ],
        out_specs=pl.BlockSpec((1, BN, F), lambda i, r: (r, i, 0)),
        out_shape=jax.ShapeDtypeStruct((R, N, F), jnp.float32),
    )(entity_embeddings, weight)
    t_flat = t.reshape(R * N, F)

    partials = _sc_scatter(t_flat, fidx, tgtp, zrows)

    out = pl.pallas_call(
        _finish_body,
        grid=(N // BF,),
        in_specs=[pl.BlockSpec((1, BF, F), lambda i: (i // (H // BF), i % (H // BF), 0))],
        out_specs=pl.BlockSpec((BF, F), lambda i: (i, 0)),
        out_shape=jax.ShapeDtypeStruct((N, F), jnp.float32),
    )(partials)
    return out


# speedup vs baseline: 30.3551x; 1.4136x over previous
import functools

import jax
import jax.numpy as jnp
from jax import lax
from jax.experimental import pallas as pl
from jax.experimental.pallas import tpu as pltpu
from jax.experimental.pallas import tpu_sc as plsc

N = 10000
EDG = 320000
F = 128
R = 8

NC, NS = 2, 16
H = N // 2
K = 128
CH = 157
EPT = K * CH
ETOT = NS * EPT
NACC = 5120
NDUMP = NACC - H
RPT = NACC // NS

BN = 1000
BF = 1000


def _mm_body(e_ref, w_ref, o_ref):
    o_ref[0] = jnp.dot(e_ref[...], w_ref[0], preferred_element_type=jnp.float32)


def _finish_body(p_ref, o_ref):
    o_ref[...] = jnp.maximum(p_ref[0], 0.0)


_mesh = plsc.VectorSubcoreMesh(
    core_axis_name="c", subcore_axis_name="s", num_cores=NC, num_subcores=NS
)


@functools.partial(
    pl.kernel,
    out_type=jax.ShapeDtypeStruct((NC, NACC, F), jnp.float32),
    mesh=_mesh,
    scratch_types=[
        pltpu.VMEM((CH, K), jnp.int32),
        pltpu.VMEM((CH, K), jnp.int32),
        pltpu.VMEM((K, F), jnp.float32),
        pltpu.VMEM((K, F), jnp.float32),
        pltpu.VMEM((K, F), jnp.float32),
        pltpu.VMEM_SHARED((NACC, F), jnp.float32),
        pltpu.SemaphoreType.DMA,
        pltpu.SemaphoreType.DMA,
        pltpu.SemaphoreType.DMA,
    ],
)
def _sc_scatter(t_hbm, fidx_hbm, tgt_hbm, zrows_hbm, out_hbm,
                fidx_v, tgt_v, r0, r1, r2, accum,
                m0, m1, m2):
    c = lax.axis_index("c")
    s = lax.axis_index("s")
    bufs = (r0, r1, r2)
    gsem = (m0, m1, m2)
    ssem = gsem

    pltpu.sync_copy(zrows_hbm, accum.at[pl.ds(s * RPT, RPT)])
    plsc.subcore_barrier()

    pltpu.sync_copy(fidx_hbm.at[s], fidx_v)
    pltpu.sync_copy(tgt_hbm.at[c, s], tgt_v)

    def gather(ch, j):
        pltpu.async_copy(t_hbm.at[fidx_v.at[ch]], bufs[j], gsem[j])

    def wait_gather(ch, j):
        pltpu.make_async_copy(t_hbm.at[fidx_v.at[ch]], bufs[j], gsem[j]).wait()

    def scatter(ch, j):
        pltpu.async_copy(bufs[j], accum.at[tgt_v.at[ch]], ssem[j], add=True)

    def wait_scatter(ch, j):
        pltpu.make_async_copy(bufs[j], accum.at[tgt_v.at[ch]], ssem[j]).wait()

    gather(0, 0)
    gather(1, 1)
    wait_gather(0, 0)
    scatter(0, 0)
    gather(2, 2)
    wait_gather(1, 1)
    scatter(1, 1)
    wait_scatter(0, 0)
    gather(3, 0)

    def body(i, carry):
        base = 3 * i + 2
        for jj in range(3):
            ch = base + jj
            j = (2 + jj) % 3
            wait_gather(ch, j)
            scatter(ch, j)
            wait_scatter(ch - 1, (j + 2) % 3)
            gather(ch + 2, (j + 2) % 3)
        return carry

    lax.fori_loop(0, (CH - 4) // 3, body, 0)

    ca = CH - 2
    wait_gather(ca, ca % 3)
    scatter(ca, ca % 3)
    wait_gather(ca + 1, (ca + 1) % 3)
    scatter(ca + 1, (ca + 1) % 3)
    for ch in range(CH - 3, CH):
        wait_scatter(ch, ch % 3)

    plsc.subcore_barrier()
    pltpu.sync_copy(accum.at[pl.ds(s * RPT, RPT)],
                    out_hbm.at[c, pl.ds(s * RPT, RPT)])


def kernel(entity_embeddings, weight, edge_index, edge_type):
    src = edge_index[0]
    tgt = edge_index[1]
    flat_idx = edge_type * N + src

    dump = H + (jnp.arange(EDG, dtype=jnp.int32) % NDUMP)
    tgt_lo = jnp.where(tgt < H, tgt, dump)
    tgt_hi = jnp.where(tgt >= H, tgt - H, dump)

    pad = ETOT - EDG
    fidx = jnp.concatenate(
        [flat_idx, jnp.zeros((pad,), jnp.int32)]).reshape(NS, CH, K)
    tgtp = jnp.concatenate([
        jnp.concatenate([tgt_lo, jnp.full((pad,), H, jnp.int32)]),
        jnp.concatenate([tgt_hi, jnp.full((pad,), H, jnp.int32)]),
    ]).reshape(NC, NS, CH, K)
    zrows = jnp.zeros((RPT, F), jnp.float32)

    t = pl.pallas_call(
        _mm_body,
        grid=(N // BN, R),
        in_specs=[
            pl.BlockSpec((BN, F), lambda i, r: (i, 0)),
            pl.BlockSpec((1, F, F), lambda i, r: (r, 0, 0)),
        ],
        out_specs=pl.BlockSpec((1, BN, F), lambda i, r: (r, i, 0)),
        out_shape=jax.ShapeDtypeStruct((R, N, F), jnp.float32),
    )(entity_embeddings, weight)
    t_flat = t.reshape(R * N, F)

    partials = _sc_scatter(t_flat, fidx, tgtp, zrows)

    out = pl.pallas_call(
        _finish_body,
        grid=(N // BF,),
        in_specs=[pl.BlockSpec((1, BF, F), lambda i: (i // (H // BF), i % (H // BF), 0))],
        out_specs=pl.BlockSpec((BF, F), lambda i: (i, 0)),
        out_shape=jax.ShapeDtypeStruct((N, F), jnp.float32),
    )(partials)
    return out


# speedup vs baseline: 40.9146x; 1.3479x over previous
import functools

import jax
import jax.numpy as jnp
from jax import lax
from jax.experimental import pallas as pl
from jax.experimental.pallas import tpu as pltpu
from jax.experimental.pallas import tpu_sc as plsc

N = 10000
EDG = 320000
F = 128
R = 8

NC, NS = 2, 16
H = N // 2
K = 128
CH = 157
NBUF = 3
EPT = K * CH
ETOT = NS * EPT
NACC = 5008
RPT = 320
LAST_Z = NACC - RPT * (NS - 1)
LAST_C = H - RPT * (NS - 1)
PADTGT = 1 << 29

BN = 1000
BF = 1000


def _mm_body(e_ref, w_ref, o_ref):
    o_ref[0] = jnp.dot(e_ref[...], w_ref[0], preferred_element_type=jnp.float32)


def _finish_body(p_ref, o_ref):
    o_ref[...] = jnp.maximum(p_ref[0], 0.0)


_mesh = plsc.VectorSubcoreMesh(
    core_axis_name="c", subcore_axis_name="s", num_cores=NC, num_subcores=NS
)


@functools.partial(
    pl.kernel,
    out_type=jax.ShapeDtypeStruct((NC, H, F), jnp.float32),
    mesh=_mesh,
    compiler_params=pltpu.CompilerParams(needs_layout_passes=False),
    scratch_types=[
        pltpu.VMEM((CH, K), jnp.int32),
        pltpu.VMEM((CH, K), jnp.int32),
        pltpu.VMEM((16,), jnp.int32),
        *([pltpu.VMEM((K, F), jnp.float32)] * NBUF),
        pltpu.VMEM_SHARED((NACC, F), jnp.float32),
        *([pltpu.SemaphoreType.DMA] * NBUF),
    ],
)
def _sc_scatter(t_hbm, fidx_hbm, tgt_hbm, zrows_hbm, out_hbm,
                fidx_v, tgt_v, cnt_v, *rest):
    bufs = rest[:NBUF]
    accum = rest[NBUF]
    gsem = rest[NBUF + 1:]
    ssem = gsem
    c = lax.axis_index("c")
    s = lax.axis_index("s")

    @pl.when(s < NS - 1)
    def _():
        pltpu.sync_copy(zrows_hbm, accum.at[pl.ds(s * RPT, RPT)])
    @pl.when(s == NS - 1)
    def _():
        pltpu.sync_copy(zrows_hbm.at[pl.ds(0, LAST_Z)],
                        accum.at[pl.ds((NS - 1) * RPT, LAST_Z)])
    plsc.subcore_barrier()

    pltpu.sync_copy(fidx_hbm.at[s], fidx_v)
    pltpu.sync_copy(tgt_hbm.at[s], tgt_v)

    lo = c * H
    lanes = jnp.arange(16, dtype=jnp.int32)
    zero16 = jnp.zeros((16,), jnp.int32)

    def comp_body(i, offv):
        row = i // (K // 16)
        col = (i % (K // 16)) * 16
        t = tgt_v[row, pl.ds(col, 16)]
        f = fidx_v[row, pl.ds(col, 16)]
        tl = t - lo
        own = (tl >= 0) & (tl < H)
        cnt = plsc.all_reduce_population_count(own)
        sel = lanes < cnt
        pos = offv + lanes
        prow = pos >> 7
        pcol = pos & (K - 1)
        plsc.store_compressed(cnt_v.at[...], f, mask=own)
        fc = cnt_v[...]
        plsc.store_scatter(fidx_v, [prow, pcol], fc, mask=sel)
        plsc.store_compressed(cnt_v.at[...], tl, mask=own)
        tc = cnt_v[...]
        plsc.store_scatter(tgt_v, [prow, pcol], tc, mask=sel)
        return offv + cnt

    offv = lax.fori_loop(0, EPT // 16, comp_body, zero16)
    cnt_v[...] = offv
    off = cnt_v[...][0]

    nch = (off + K - 1) // K
    end = nch * K
    zeros16 = jnp.zeros((16,), jnp.int32)
    dumps16 = jnp.full((16,), H, jnp.int32)
    for b in range(K // 16):
        pos = off + b * 16 + lanes
        m = pos < end
        prow = pos >> 7
        pcol = pos & (K - 1)
        plsc.store_scatter(fidx_v, [prow, pcol], zeros16, mask=m)
        plsc.store_scatter(tgt_v, [prow, pcol], dumps16, mask=m)

    def gather(ch, j):
        pltpu.async_copy(t_hbm.at[fidx_v.at[ch]], bufs[j], gsem[j])

    def wait_gather(ch, j):
        pltpu.make_async_copy(t_hbm.at[fidx_v.at[ch]], bufs[j], gsem[j]).wait()

    def scatter(ch, j):
        pltpu.async_copy(bufs[j], accum.at[tgt_v.at[ch]], ssem[j], add=True)

    def wait_scatter(ch, j):
        pltpu.make_async_copy(bufs[j], accum.at[tgt_v.at[ch]], ssem[j]).wait()

    def pipe_step(ch, j):
        @pl.when(ch < nch)
        def _():
            wait_gather(ch, j)
            scatter(ch, j)
        d = ch - (NBUF - 2)
        if not (isinstance(d, int) and d < 0):
            @pl.when(d < nch)
            def _():
                wait_scatter(d, (j + 2) % NBUF)
        g = ch + 2
        @pl.when(g < nch)
        def _():
            gather(g, (j + 2) % NBUF)

    for ch in range(2):
        @pl.when(ch < nch)
        def _():
            gather(ch, ch % NBUF)
    for ch in range(2):
        pipe_step(ch, ch % NBUF)

    def body(i, carry):
        base = NBUF * i + 2
        for jj in range(NBUF):
            pipe_step(base + jj, (2 + jj) % NBUF)
        return carry

    lax.fori_loop(0, (CH - 4) // NBUF, body, 0)

    for ch in range(CH - 2, CH + 1):
        pipe_step(ch, ch % NBUF)

    plsc.subcore_barrier()
    @pl.when(s < NS - 1)
    def _():
        pltpu.sync_copy(accum.at[pl.ds(s * RPT, RPT)],
                        out_hbm.at[c, pl.ds(s * RPT, RPT)])
    @pl.when(s == NS - 1)
    def _():
        pltpu.sync_copy(accum.at[pl.ds((NS - 1) * RPT, LAST_C)],
                        out_hbm.at[c, pl.ds((NS - 1) * RPT, LAST_C)])


def kernel(entity_embeddings, weight, edge_index, edge_type):
    src = edge_index[0]
    tgt = edge_index[1]
    flat_idx = edge_type * N + src

    pad = ETOT - EDG
    fidx = jnp.concatenate(
        [flat_idx, jnp.zeros((pad,), jnp.int32)]).reshape(NS, CH, K)
    tgtp = jnp.concatenate(
        [tgt, jnp.full((pad,), PADTGT, jnp.int32)]).reshape(NS, CH, K)
    zrows = jnp.zeros((RPT, F), jnp.float32)

    t = pl.pallas_call(
        _mm_body,
        grid=(N // BN, R),
        in_specs=[
            pl.BlockSpec((BN, F), lambda i, r: (i, 0)),
            pl.BlockSpec((1, F, F), lambda i, r: (r, 0, 0)),
        ],
        out_specs=pl.BlockSpec((1, BN, F), lambda i, r: (r, i, 0)),
        out_shape=jax.ShapeDtypeStruct((R, N, F), jnp.float32),
    )(entity_embeddings, weight)
    t_flat = t.reshape(R * N, F)

    partials = _sc_scatter(t_flat, fidx, tgtp, zrows)

    out = pl.pallas_call(
        _finish_body,
        grid=(N // BF,),
        in_specs=[pl.BlockSpec((1, BF, F),
                               lambda i: (i // (H // BF), i % (H // BF), 0))],
        out_specs=pl.BlockSpec((BF, F), lambda i: (i, 0)),
        out_shape=jax.ShapeDtypeStruct((N, F), jnp.float32),
    )(partials)
    return out
